# baseline (speedup 1.0000x reference)
import jax
import jax.numpy as jnp
from jax.experimental import pallas as pl
from jax.experimental.pallas import tpu as pltpu

_K = 8192
_D = 256
_B = 32768
_BR = 256
_NB = _B // _BR
_CC = 0.25


def _vq_body(x_ref, e_ref, idx_ref, enc_ref, qst_ref, cnt_ref, loss_ref):
    i = pl.program_id(0)
    x = x_ref[...]
    e = e_ref[...]
    a = jnp.sum(x * x, axis=1, keepdims=True)
    b = jnp.sum(e * e, axis=1)
    c = jax.lax.dot_general(
        x, e, (((1,), (1,)), ((), ())),
        preferred_element_type=jnp.float32)
    d = (a + b) - 2.0 * c
    dmin = jnp.min(d, axis=1, keepdims=True)
    col = jax.lax.broadcasted_iota(jnp.int32, (_BR, _K), 1)
    idx = jnp.min(jnp.where(d == dmin, col, _K), axis=1).astype(jnp.int32)
    idx_ref[0, 0, :] = idx
    onehot = (col == idx[:, None]).astype(jnp.float32)
    enc_ref[...] = onehot
    q = jax.lax.dot_general(
        onehot, e, (((1,), (0,)), ((), ())),
        preferred_element_type=jnp.float32)
    qst_ref[...] = x + (q - x)
    pcnt = jnp.sum(onehot, axis=0, keepdims=True)
    ploss = jnp.sum(dmin.reshape(2, _BR // 2), axis=0, keepdims=True)

    @pl.when(i == 0)
    def _():
        cnt_ref[...] = pcnt
        loss_ref[...] = ploss

    @pl.when(i > 0)
    def _():
        cnt_ref[...] += pcnt
        loss_ref[...] += ploss


_VQ_GRID = (_NB,)
_VQ_IN_SPECS = [
    pl.BlockSpec((_BR, _D), lambda i: (i, 0)),
    pl.BlockSpec((_K, _D), lambda i: (0, 0)),
]
_VQ_OUT_SPECS = [
    pl.BlockSpec((1, 1, _BR), lambda i: (i, 0, 0)),
    pl.BlockSpec((_BR, _K), lambda i: (i, 0)),
    pl.BlockSpec((_BR, _D), lambda i: (i, 0)),
    pl.BlockSpec((1, _K), lambda i: (0, 0)),
    pl.BlockSpec((1, _BR // 2), lambda i: (0, 0)),
]
_VQ_OUT_SHAPE = [
    jax.ShapeDtypeStruct((_NB, 1, _BR), jnp.int32),
    jax.ShapeDtypeStruct((_B, _K), jnp.float32),
    jax.ShapeDtypeStruct((_B, _D), jnp.float32),
    jax.ShapeDtypeStruct((1, _K), jnp.float32),
    jax.ShapeDtypeStruct((1, _BR // 2), jnp.float32),
]


def kernel(inputs, label, embedding_weight):
    idx3, enc, qst, cnt, losspart = pl.pallas_call(
        _vq_body,
        grid=_VQ_GRID,
        in_specs=_VQ_IN_SPECS,
        out_specs=_VQ_OUT_SPECS,
        out_shape=_VQ_OUT_SHAPE,
    )(inputs, embedding_weight)
    a = jnp.sum(losspart) / (_B * _D)
    loss = a + _CC * a
    p = cnt[0] / _B
    perplexity = jnp.exp(-jnp.sum(p * jnp.log(p + 1e-10)))
    return (loss, qst, perplexity, enc)


# speedup vs baseline: 1.1749x; 1.1749x over previous
import functools

import jax
import jax.numpy as jnp
from jax import lax
from jax.experimental import pallas as pl
from jax.experimental.pallas import tpu as pltpu
from jax.experimental.pallas import tpu_sc as plsc

_K = 8192
_D = 256
_B = 32768
_BR = 256
_NB = _B // _BR
_CC = 0.25


def _vq_body(x_ref, e_ref, idx_ref, enc_ref, cnt_ref, loss_ref):
    i = pl.program_id(0)
    x = x_ref[...]
    e = e_ref[...]
    a = jnp.sum(x * x, axis=1, keepdims=True)
    b = jnp.sum(e * e, axis=1)
    c = jax.lax.dot_general(
        x, e, (((1,), (1,)), ((), ())),
        preferred_element_type=jnp.float32)
    d = (a + b) - 2.0 * c
    dmin = jnp.min(d, axis=1, keepdims=True)
    col = jax.lax.broadcasted_iota(jnp.int32, (_BR, _K), 1)
    idx = jnp.min(jnp.where(d == dmin, col, _K), axis=1).astype(jnp.int32)
    idx_ref[0, 0, :] = idx
    onehot = (col == idx[:, None]).astype(jnp.float32)
    enc_ref[...] = onehot
    pcnt = jnp.sum(onehot, axis=0, keepdims=True)
    ploss = jnp.sum(dmin.reshape(2, _BR // 2), axis=0, keepdims=True)

    @pl.when(i == 0)
    def _():
        cnt_ref[...] = pcnt
        loss_ref[...] = ploss

    @pl.when(i > 0)
    def _():
        cnt_ref[...] += pcnt
        loss_ref[...] += ploss


_vq_call = pl.pallas_call(
    _vq_body,
    grid=(_NB,),
    in_specs=[
        pl.BlockSpec((_BR, _D), lambda i: (i, 0)),
        pl.BlockSpec((_K, _D), lambda i: (0, 0)),
    ],
    out_specs=[
        pl.BlockSpec((1, 1, _BR), lambda i: (i, 0, 0)),
        pl.BlockSpec((_BR, _K), lambda i: (i, 0)),
        pl.BlockSpec((1, _K), lambda i: (0, 0)),
        pl.BlockSpec((1, _BR // 2), lambda i: (0, 0)),
    ],
    out_shape=[
        jax.ShapeDtypeStruct((_NB, 1, _BR), jnp.int32),
        jax.ShapeDtypeStruct((_B, _K), jnp.float32),
        jax.ShapeDtypeStruct((1, _K), jnp.float32),
        jax.ShapeDtypeStruct((1, _BR // 2), jnp.float32),
    ],
)

_SC_INFO = plsc.get_sparse_core_info()
_NC = _SC_INFO.num_cores
_NS = _SC_INFO.num_subcores
_NW = _NC * _NS
_BPW = _B // _NW
_CH = 128
_NCH = _BPW // _CH


@functools.partial(
    pl.kernel,
    mesh=plsc.VectorSubcoreMesh(core_axis_name="c", subcore_axis_name="s"),
    out_type=jax.ShapeDtypeStruct((_B, _D), jnp.float32),
    scratch_types=[
        pltpu.VMEM((_CH,), jnp.int32),
        pltpu.VMEM((_CH, _D), jnp.float32),
        pltpu.SemaphoreType.DMA,
    ],
)
def _sc_gather(idx_hbm, table_hbm, out_hbm, idx_v, rows_v, sem):
    wid = lax.axis_index("s") * _NC + lax.axis_index("c")
    base = wid * _BPW
    for ci in range(_NCH):
        off = base + ci * _CH
        pltpu.sync_copy(idx_hbm.at[pl.ds(off, _CH)], idx_v)
        pltpu.async_copy(table_hbm.at[idx_v], rows_v, sem).wait()
        pltpu.sync_copy(rows_v, out_hbm.at[pl.ds(off, _CH)])


def kernel(inputs, label, embedding_weight):
    idx3, enc, cnt, losspart = _vq_call(inputs, embedding_weight)
    q = _sc_gather(idx3.reshape(_B), embedding_weight)
    a = jnp.sum(losspart) / (_B * _D)
    loss = a + _CC * a
    p = cnt[0] / _B
    perplexity = jnp.exp(-jnp.sum(p * jnp.log(p + 1e-10)))
    return (loss, q, perplexity, enc)


# speedup vs baseline: 1.4031x; 1.1942x over previous
import functools

import jax
import jax.numpy as jnp
from jax import lax
from jax.experimental import pallas as pl
from jax.experimental.pallas import tpu as pltpu
from jax.experimental.pallas import tpu_sc as plsc

_K = 8192
_D = 256
_B = 32768
_BR = 256
_NB = _B // _BR
_CC = 0.25


def _vq_body(x_ref, e_ref, idx_ref, enc_ref, cnt_ref, loss_ref, bsq_ref):
    i = pl.program_id(0)

    @pl.when(i == 0)
    def _():
        e0 = e_ref[...]
        bsq_ref[...] = jnp.sum(e0 * e0, axis=1).reshape(1, _K)

    x = x_ref[...]
    a = jnp.sum(x * x, axis=1, keepdims=True)
    b = bsq_ref[...]
    c = jax.lax.dot_general(
        x, e_ref[...], (((1,), (1,)), ((), ())),
        preferred_element_type=jnp.float32)
    d = (a + b) - 2.0 * c
    dmin = jnp.min(d, axis=1, keepdims=True)
    idx = jnp.argmin(d, axis=1).astype(jnp.int32)
    idx_ref[0, 0, :] = idx
    col = jax.lax.broadcasted_iota(jnp.int32, (_BR, _K), 1)
    onehot = (col == idx[:, None]).astype(jnp.float32)
    enc_ref[...] = onehot
    pcnt = jnp.sum(onehot, axis=0, keepdims=True)
    ploss = jnp.sum(dmin.reshape(2, _BR // 2), axis=0, keepdims=True)

    @pl.when(i == 0)
    def _():
        cnt_ref[...] = pcnt
        loss_ref[...] = ploss

    @pl.when(i > 0)
    def _():
        cnt_ref[...] += pcnt
        loss_ref[...] += ploss


_vq_call = pl.pallas_call(
    _vq_body,
    grid=(_NB,),
    in_specs=[
        pl.BlockSpec((_BR, _D), lambda i: (i, 0)),
        pl.BlockSpec((_K, _D), lambda i: (0, 0)),
    ],
    out_specs=[
        pl.BlockSpec((1, 1, _BR), lambda i: (i, 0, 0)),
        pl.BlockSpec((_BR, _K), lambda i: (i, 0)),
        pl.BlockSpec((1, _K), lambda i: (0, 0)),
        pl.BlockSpec((1, _BR // 2), lambda i: (0, 0)),
    ],
    out_shape=[
        jax.ShapeDtypeStruct((_NB, 1, _BR), jnp.int32),
        jax.ShapeDtypeStruct((_B, _K), jnp.float32),
        jax.ShapeDtypeStruct((1, _K), jnp.float32),
        jax.ShapeDtypeStruct((1, _BR // 2), jnp.float32),
    ],
    scratch_shapes=[pltpu.VMEM((1, _K), jnp.float32)],
)

_SC_INFO = plsc.get_sparse_core_info()
_NC = _SC_INFO.num_cores
_NS = _SC_INFO.num_subcores
_NW = _NC * _NS
_BPW = _B // _NW
_CH = 128
_NCH = _BPW // _CH


@functools.partial(
    pl.kernel,
    mesh=plsc.VectorSubcoreMesh(core_axis_name="c", subcore_axis_name="s"),
    out_type=jax.ShapeDtypeStruct((_B, _D), jnp.float32),
    scratch_types=[
        pltpu.VMEM((_CH,), jnp.int32),
        pltpu.VMEM((_CH, _D), jnp.float32),
        pltpu.SemaphoreType.DMA,
    ],
)
def _sc_gather(idx_hbm, table_hbm, out_hbm, idx_v, rows_v, sem):
    wid = lax.axis_index("s") * _NC + lax.axis_index("c")
    base = wid * _BPW
    for ci in range(_NCH):
        off = base + ci * _CH
        pltpu.sync_copy(idx_hbm.at[pl.ds(off, _CH)], idx_v)
        pltpu.async_copy(table_hbm.at[idx_v], rows_v, sem).wait()
        pltpu.sync_copy(rows_v, out_hbm.at[pl.ds(off, _CH)])


def kernel(inputs, label, embedding_weight):
    idx3, enc, cnt, losspart = _vq_call(inputs, embedding_weight)
    q = _sc_gather(idx3.reshape(_B), embedding_weight)
    a = jnp.sum(losspart) / (_B * _D)
    loss = a + _CC * a
    p = cnt[0] / _B
    perplexity = jnp.exp(-jnp.sum(p * jnp.log(p + 1e-10)))
    return (loss, q, perplexity, enc)


# speedup vs baseline: 1.4065x; 1.0025x over previous
import functools

import jax
import jax.numpy as jnp
from jax import lax
from jax.experimental import pallas as pl
from jax.experimental.pallas import tpu as pltpu
from jax.experimental.pallas import tpu_sc as plsc

_K = 8192
_D = 256
_B = 32768
_BR = 256
_NB = _B // _BR
_CC = 0.25


def _vq_body(x_ref, e_ref, idx_ref, enc_ref, cnt_ref, loss_ref, bsq_ref):
    i = pl.program_id(0)

    @pl.when(i == 0)
    def _():
        e0 = e_ref[...]
        bsq_ref[...] = jnp.sum(e0 * e0, axis=1).reshape(1, _K)

    x = x_ref[...]
    a = jnp.sum(x * x, axis=1, keepdims=True)
    b = bsq_ref[...]
    c = jax.lax.dot_general(
        x, e_ref[...], (((1,), (1,)), ((), ())),
        preferred_element_type=jnp.float32)
    d = (a + b) - 2.0 * c
    dmin = jnp.min(d, axis=1, keepdims=True)
    col = jax.lax.broadcasted_iota(jnp.int32, (_BR, _K), 1)
    idx = jnp.min(jnp.where(d == dmin, col, _K), axis=1).astype(jnp.int32)
    idx_ref[0, 0, :] = idx
    onehot = (col == idx[:, None]).astype(jnp.float32)
    enc_ref[...] = onehot
    pcnt = jnp.sum(onehot, axis=0, keepdims=True)
    ploss = jnp.sum(dmin.reshape(2, _BR // 2), axis=0, keepdims=True)

    @pl.when(i == 0)
    def _():
        cnt_ref[...] = pcnt
        loss_ref[...] = ploss

    @pl.when(i > 0)
    def _():
        cnt_ref[...] += pcnt
        loss_ref[...] += ploss


_vq_call = pl.pallas_call(
    _vq_body,
    grid=(_NB,),
    in_specs=[
        pl.BlockSpec((_BR, _D), lambda i: (i, 0)),
        pl.BlockSpec((_K, _D), lambda i: (0, 0)),
    ],
    out_specs=[
        pl.BlockSpec((1, 1, _BR), lambda i: (i, 0, 0)),
        pl.BlockSpec((_BR, _K), lambda i: (i, 0)),
        pl.BlockSpec((1, _K), lambda i: (0, 0)),
        pl.BlockSpec((1, _BR // 2), lambda i: (0, 0)),
    ],
    out_shape=[
        jax.ShapeDtypeStruct((_NB, 1, _BR), jnp.int32),
        jax.ShapeDtypeStruct((_B, _K), jnp.float32),
        jax.ShapeDtypeStruct((1, _K), jnp.float32),
        jax.ShapeDtypeStruct((1, _BR // 2), jnp.float32),
    ],
    scratch_shapes=[pltpu.VMEM((1, _K), jnp.float32)],
)

_SC_INFO = plsc.get_sparse_core_info()
_NC = _SC_INFO.num_cores
_NS = _SC_INFO.num_subcores
_NW = _NC * _NS
_BPW = _B // _NW
_CH = 128
_NCH = _BPW // _CH


@functools.partial(
    pl.kernel,
    mesh=plsc.VectorSubcoreMesh(core_axis_name="c", subcore_axis_name="s"),
    out_type=jax.ShapeDtypeStruct((_B, _D), jnp.float32),
    scratch_types=[
        pltpu.VMEM((_CH,), jnp.int32),
        pltpu.VMEM((_CH, _D), jnp.float32),
        pltpu.SemaphoreType.DMA,
    ],
)
def _sc_gather(idx_hbm, table_hbm, out_hbm, idx_v, rows_v, sem):
    wid = lax.axis_index("s") * _NC + lax.axis_index("c")
    base = wid * _BPW
    for ci in range(_NCH):
        off = base + ci * _CH
        pltpu.sync_copy(idx_hbm.at[pl.ds(off, _CH)], idx_v)
        pltpu.async_copy(table_hbm.at[idx_v], rows_v, sem).wait()
        pltpu.sync_copy(rows_v, out_hbm.at[pl.ds(off, _CH)])


def kernel(inputs, label, embedding_weight):
    idx3, enc, cnt, losspart = _vq_call(inputs, embedding_weight)
    q = _sc_gather(idx3.reshape(_B), embedding_weight)
    a = jnp.sum(losspart) / (_B * _D)
    loss = a + _CC * a
    p = cnt[0] / _B
    perplexity = jnp.exp(-jnp.sum(p * jnp.log(p + 1e-10)))
    return (loss, q, perplexity, enc)


# speedup vs baseline: 1.5823x; 1.1250x over previous
import functools

import jax
import jax.numpy as jnp
from jax import lax
from jax.experimental import pallas as pl
from jax.experimental.pallas import tpu as pltpu
from jax.experimental.pallas import tpu_sc as plsc

_K = 8192
_D = 256
_B = 32768
_BR = 256
_NB = _B // _BR
_CC = 0.25


def _vq_body(x_ref, e_ref, idx_ref, enc_ref, cnt_ref, loss_ref, bsq_ref):
    i = pl.program_id(0)

    @pl.when(i == 0)
    def _():
        e0 = e_ref[...]
        bsq_ref[...] = jnp.sum(e0 * e0, axis=1).reshape(1, _K)

    x = x_ref[...]
    a = jnp.sum(x * x, axis=1, keepdims=True)
    b = bsq_ref[...]
    c = jax.lax.dot_general(
        x, e_ref[...], (((1,), (1,)), ((), ())),
        preferred_element_type=jnp.float32)
    d = (a + b) - 2.0 * c
    dmin = jnp.min(d, axis=1, keepdims=True)
    col = jax.lax.broadcasted_iota(jnp.int32, (_BR, _K), 1)
    idx = jnp.min(jnp.where(d == dmin, col, _K), axis=1).astype(jnp.int32)
    idx_ref[0, 0, :] = idx
    onehot = (col == idx[:, None]).astype(jnp.float32)
    enc_ref[...] = onehot
    ones_row = jnp.ones((1, _BR), dtype=jnp.float32)
    pcnt = jax.lax.dot_general(
        ones_row, onehot, (((1,), (0,)), ((), ())),
        preferred_element_type=jnp.float32)
    ploss = jnp.sum(dmin.reshape(2, _BR // 2), axis=0, keepdims=True)

    @pl.when(i == 0)
    def _():
        cnt_ref[...] = pcnt
        loss_ref[...] = ploss

    @pl.when(i > 0)
    def _():
        cnt_ref[...] += pcnt
        loss_ref[...] += ploss


_vq_call = pl.pallas_call(
    _vq_body,
    grid=(_NB,),
    in_specs=[
        pl.BlockSpec((_BR, _D), lambda i: (i, 0)),
        pl.BlockSpec((_K, _D), lambda i: (0, 0)),
    ],
    out_specs=[
        pl.BlockSpec((1, 1, _BR), lambda i: (i, 0, 0)),
        pl.BlockSpec((_BR, _K), lambda i: (i, 0)),
        pl.BlockSpec((1, _K), lambda i: (0, 0)),
        pl.BlockSpec((1, _BR // 2), lambda i: (0, 0)),
    ],
    out_shape=[
        jax.ShapeDtypeStruct((_NB, 1, _BR), jnp.int32),
        jax.ShapeDtypeStruct((_B, _K), jnp.float32),
        jax.ShapeDtypeStruct((1, _K), jnp.float32),
        jax.ShapeDtypeStruct((1, _BR // 2), jnp.float32),
    ],
    scratch_shapes=[pltpu.VMEM((1, _K), jnp.float32)],
)

_SC_INFO = plsc.get_sparse_core_info()
_NC = _SC_INFO.num_cores
_NS = _SC_INFO.num_subcores
_NW = _NC * _NS
_BPW = _B // _NW
_CH = 128
_NCH = _BPW // _CH


@functools.partial(
    pl.kernel,
    mesh=plsc.VectorSubcoreMesh(core_axis_name="c", subcore_axis_name="s"),
    out_type=jax.ShapeDtypeStruct((_B, _D), jnp.float32),
    scratch_types=[
        pltpu.VMEM((_CH,), jnp.int32),
        pltpu.VMEM((_CH, _D), jnp.float32),
        pltpu.SemaphoreType.DMA,
    ],
)
def _sc_gather(idx_hbm, table_hbm, out_hbm, idx_v, rows_v, sem):
    wid = lax.axis_index("s") * _NC + lax.axis_index("c")
    base = wid * _BPW
    for ci in range(_NCH):
        off = base + ci * _CH
        pltpu.sync_copy(idx_hbm.at[pl.ds(off, _CH)], idx_v)
        pltpu.async_copy(table_hbm.at[idx_v], rows_v, sem).wait()
        pltpu.sync_copy(rows_v, out_hbm.at[pl.ds(off, _CH)])


def kernel(inputs, label, embedding_weight):
    idx3, enc, cnt, losspart = _vq_call(inputs, embedding_weight)
    q = _sc_gather(idx3.reshape(_B), embedding_weight)
    a = jnp.sum(losspart) / (_B * _D)
    loss = a + _CC * a
    p = cnt[0] / _B
    perplexity = jnp.exp(-jnp.sum(p * jnp.log(p + 1e-10)))
    return (loss, q, perplexity, enc)
